# baseline (speedup 1.0000x reference)
import functools

import jax
import jax.numpy as jnp
from jax import lax
from jax.experimental import pallas as pl
from jax.experimental.pallas import tpu as pltpu
from jax.experimental.pallas import tpu_sc as plsc

FEATURE = 1024
QUEUE = 65536
BATCH = 4096
NC, NS = 2, 16
NW = NC * NS
WROWS = FEATURE // NW
CC = 1024
NKC = BATCH // CC
NCH = QUEUE // CC


def _chunk_in(tkeys_ref, queue_ref, r0, i, buf, sem):
    src = tkeys_ref if i < NKC else queue_ref
    return pltpu.make_async_copy(
        src.at[pl.ds(r0, WROWS), pl.ds(i * CC, CC)], buf, sem)


def _chunk_out(out_ref, r0, i, buf, sem):
    return pltpu.make_async_copy(
        buf, out_ref.at[pl.ds(r0, WROWS), pl.ds(i * CC, CC)], sem)


def _sc_body(tkeys_ref, queue_ref, out_ref,
             buf0, buf1, isem0, isem1, osem0, osem1):
    wid = lax.axis_index("s") * NC + lax.axis_index("c")
    r0 = wid * WROWS

    bufs = (buf0, buf1)
    isems = (isem0, isem1)
    osems = (osem0, osem1)

    _chunk_in(tkeys_ref, queue_ref, r0, 0, buf0, isem0).start()
    for i in range(NCH):
        s = i % 2
        _chunk_in(tkeys_ref, queue_ref, r0, i, bufs[s], isems[s]).wait()
        if i >= 1:
            _chunk_out(out_ref, r0, i - 1, bufs[s ^ 1], osems[s ^ 1]).wait()
        if i + 1 < NCH:
            _chunk_in(tkeys_ref, queue_ref, r0, i + 1, bufs[s ^ 1], isems[s ^ 1]).start()
        _chunk_out(out_ref, r0, i, bufs[s], osems[s]).start()
    _chunk_out(out_ref, r0, NCH - 1, bufs[(NCH - 1) % 2], osems[(NCH - 1) % 2]).wait()


def _t_body(keys_ref, tk_ref):
    tk_ref[...] = keys_ref[...].T


def _transpose_tc(keys):
    return pl.pallas_call(
        _t_body,
        grid=(BATCH // CC,),
        in_specs=[pl.BlockSpec((CC, FEATURE), lambda j: (j, 0))],
        out_specs=pl.BlockSpec((FEATURE, CC), lambda j: (0, j)),
        out_shape=jax.ShapeDtypeStruct((FEATURE, BATCH), jnp.float32),
    )(keys)


def kernel(keys, queue):
    tkeys = _transpose_tc(keys)
    mesh = plsc.VectorSubcoreMesh(core_axis_name="c", subcore_axis_name="s")
    k = functools.partial(
        pl.kernel,
        out_type=jax.ShapeDtypeStruct((FEATURE, QUEUE), jnp.float32),
        mesh=mesh,
        scratch_types=[
            pltpu.VMEM((WROWS, CC), jnp.float32),
            pltpu.VMEM((WROWS, CC), jnp.float32),
            pltpu.SemaphoreType.DMA,
            pltpu.SemaphoreType.DMA,
            pltpu.SemaphoreType.DMA,
            pltpu.SemaphoreType.DMA,
        ],
    )(_sc_body)
    return k(tkeys, queue)


# speedup vs baseline: 1.3454x; 1.3454x over previous
import jax
import jax.numpy as jnp
from jax.experimental import pallas as pl
from jax.experimental.pallas import tpu as pltpu

FEATURE = 1024
QUEUE = 65536
BATCH = 4096
C = 1024
K = 4
NB = (QUEUE - BATCH) // C
T = NB // K
TCH = 4
TR = BATCH // TCH


def _bulk_in(queue_ref, sbuf, isems, t, k):
    c = t * K + k
    return pltpu.make_async_copy(
        queue_ref.at[:, pl.ds(BATCH + c * C, C)],
        sbuf.at[k, t % 2], isems.at[k, t % 2])


def _bulk_out(out_ref, sbuf, osems, t, k):
    c = t * K + k
    return pltpu.make_async_copy(
        sbuf.at[k, t % 2],
        out_ref.at[:, pl.ds(BATCH + c * C, C)], osems.at[k, t % 2])


def _t_out(out_ref, tbuf, tsems, r):
    return pltpu.make_async_copy(
        tbuf.at[r % 2],
        out_ref.at[:, pl.ds(r * TR, TR)], tsems.at[r % 2])


def _body(keys_ref, queue_ref, out_ref, kbuf, tbuf, sbuf,
          ksem, tsems, isems, osems):
    kfetch = pltpu.make_async_copy(keys_ref, kbuf, ksem)
    kfetch.start()
    for k in range(K):
        _bulk_in(queue_ref, sbuf, isems, 0, k).start()
    kfetch.wait()
    for t in range(T):
        for k in range(K):
            _bulk_in(queue_ref, sbuf, isems, t, k).wait()
            if t >= 1:
                _bulk_out(out_ref, sbuf, osems, t - 1, k).wait()
            if t + 1 < T:
                _bulk_in(queue_ref, sbuf, isems, t + 1, k).start()
            _bulk_out(out_ref, sbuf, osems, t, k).start()
        if t < TCH:
            if t >= 2:
                _t_out(out_ref, tbuf, tsems, t - 2).wait()
            tbuf[t % 2] = kbuf[t * TR:(t + 1) * TR, :].T
            _t_out(out_ref, tbuf, tsems, t).start()
    for k in range(K):
        _bulk_out(out_ref, sbuf, osems, T - 1, k).wait()
    for r in (TCH - 2, TCH - 1):
        _t_out(out_ref, tbuf, tsems, r).wait()


def kernel(keys, queue):
    return pl.pallas_call(
        _body,
        in_specs=[
            pl.BlockSpec(memory_space=pltpu.MemorySpace.HBM),
            pl.BlockSpec(memory_space=pltpu.MemorySpace.HBM),
        ],
        out_specs=pl.BlockSpec(memory_space=pltpu.MemorySpace.HBM),
        out_shape=jax.ShapeDtypeStruct((FEATURE, QUEUE), jnp.float32),
        scratch_shapes=[
            pltpu.VMEM((BATCH, FEATURE), jnp.float32),
            pltpu.VMEM((2, FEATURE, TR), jnp.float32),
            pltpu.VMEM((K, 2, FEATURE, C), jnp.float32),
            pltpu.SemaphoreType.DMA,
            pltpu.SemaphoreType.DMA((2,)),
            pltpu.SemaphoreType.DMA((K, 2)),
            pltpu.SemaphoreType.DMA((K, 2)),
        ],
    )(keys, queue)
